# baseline (speedup 1.0000x reference)
import jax
import jax.numpy as jnp
from jax import lax
from jax.experimental import pallas as pl
from jax.experimental.pallas import tpu as pltpu
from jax.experimental.pallas import tpu_sc as plsc

N = 10000
E = 320000
L = 16
F = 128
H = 8
FO = 16
V = 1000
NLANES = 16
NC, NS = 2, 16
NPAD = 10240
NCH = NPAD // 64
EC = 80
CPT = 125
ERPAD = 4032
NSUP = 42
DH = F + 16
DA = F + 8
DT = 2 * F


def _dense_body(lf, re_, le, wp, asf, adf, fA, rA, lA):
    w = wp[...]
    a_s = asf[...]
    a_d = adf[...]
    j = lax.broadcasted_iota(jnp.int32, (F, L), 0)
    l = lax.broadcasted_iota(jnp.int32, (F, L), 1)
    b16 = ((j % H) == (l % H)).astype(jnp.float32)
    we_s = jnp.dot((w * a_s), b16, preferred_element_type=jnp.float32)
    we_d = jnp.dot((w * a_d), b16, preferred_element_type=jnp.float32)
    wext = jnp.concatenate(
        [w, we_s, we_d, jnp.zeros((F, DT - F - 2 * L), jnp.float32)], axis=1)
    f = jnp.dot(lf[...], wext[:L - 2], preferred_element_type=jnp.float32)
    fA[...] = jnp.concatenate(
        [f, jnp.zeros((NPAD - N, DT), jnp.float32)], axis=0)
    rA[...] = jnp.dot(re_[...], wext[L - 2:], preferred_element_type=jnp.float32)
    lA[...] = jnp.dot(le[...], wext[L - 2:], preferred_element_type=jnp.float32)


def _node_body(fA, rA, lA, i0_2d, i1_2d, eix, ht_out, ed_out, src_p, dst_p,
               fb0, fb1, rb0, rb1, lb0, lb1, i0b0, i0b1, i1b0, i1b1,
               htb0, htb1, edb0, edb1, gf, gr, gl, wh, we, ge_):
    cid = lax.axis_index("c")
    sid = lax.axis_index("s")
    wid = sid * NC + cid

    erows = E // EC // 32
    r0 = wid * erows
    e1 = pltpu.async_copy(eix.at[pl.ds(r0, erows)],
                          src_p.at[pl.ds(r0, erows)], ge_.at[0])
    e2 = pltpu.async_copy(eix.at[pl.ds(E // EC + r0, erows)],
                          dst_p.at[pl.ds(r0, erows)], ge_.at[1])
    fbs, rbs, lbs = (fb0, fb1), (rb0, rb1), (lb0, lb1)
    i0s, i1s = (i0b0, i0b1), (i1b0, i1b1)
    hts, eds = (htb0, htb1), (edb0, edb1)

    def issue(t, b):
        c = wid + t * 32
        pltpu.sync_copy(i0_2d.at[c], i0s[b])
        pltpu.sync_copy(i1_2d.at[c], i1s[b])
        pltpu.async_copy(fA.at[pl.ds(c * 64, 64)], fbs[b], gf.at[b])
        pltpu.async_copy(rA.at[i0s[b]], rbs[b], gr.at[b])
        pltpu.async_copy(lA.at[i1s[b]], lbs[b], gl.at[b])

    issue(0, 0)
    for t in range(5):
        b = t % 2
        if t < 4:
            issue(t + 1, 1 - b)
        pltpu.make_async_copy(fA.at[pl.ds(0, 64)], fbs[b], gf.at[b]).wait()
        pltpu.make_async_copy(rA.at[i0s[b]], rbs[b], gr.at[b]).wait()
        pltpu.make_async_copy(lA.at[i1s[b]], lbs[b], gl.at[b]).wait()
        if t >= 2:
            pltpu.make_async_copy(hts[b], ht_out.at[pl.ds(0, 64)],
                                  wh.at[b]).wait()
            pltpu.make_async_copy(eds[b], ed_out.at[pl.ds(0, 64)],
                                  we.at[b]).wait()
        fb, rb, lb, htb, edb = fbs[b], rbs[b], lbs[b], hts[b], eds[b]

        @plsc.parallel_loop(0, 64, unroll=2)
        def _(n):
            for g in range(H):
                s = pl.ds(g * 16, 16)
                htb[n, s] = fb[n, s] + rb[n, s] + lb[n, s]
            se = pl.ds(F, 16)
            htb[n, pl.ds(F, 16)] = fb[n, se] + rb[n, se] + lb[n, se]
            sd = pl.ds(F + 16, 16)
            edb[n, :] = fb[n, sd] + rb[n, sd] + lb[n, sd]

        base = (wid + t * 32) * 64
        pltpu.async_copy(htb, ht_out.at[pl.ds(base, 64)], wh.at[b])
        pltpu.async_copy(edb, ed_out.at[pl.ds(base, 64)], we.at[b])
    for b in range(2):
        pltpu.make_async_copy(hts[b], ht_out.at[pl.ds(0, 64)],
                              wh.at[b]).wait()
        pltpu.make_async_copy(eds[b], ed_out.at[pl.ds(0, 64)],
                              we.at[b]).wait()
    e1.wait()
    e2.wait()


def _edge_body(ht, ed, src2d, dst2d, part_out, hb0, hb1, hb2, db0, db1,
               sb0, sb1, dk0, dk1, acc, gh, ge, ss, gi):
    cid = lax.axis_index("c")
    sid = lax.axis_index("s")
    wid = sid * NC + cid
    ntile = jnp.minimum(E // EC - wid * CPT, CPT)
    rowbase = wid * CPT
    hbs = (hb0, hb1, hb2)
    dbs = (db0, db1)
    sbs = (sb0, sb1)
    dks = (dk0, dk1)

    zrow = jnp.zeros((NLANES,), jnp.float32)

    def zero_row(n, _):
        for g in range(DH // 16):
            hb0[n, pl.ds(g * 16, 16)] = zrow
        return 0

    lax.fori_loop(0, EC, zero_row, 0)
    for k in range(NPAD // NS // EC):
        pltpu.sync_copy(hb0, acc.at[pl.ds(sid * 640 + k * EC, EC)])
    plsc.subcore_barrier()

    def load_block(s, b):
        row0 = rowbase + 3 * s
        pltpu.async_copy(src2d.at[pl.ds(row0, 3)], sbs[b], gi.at[2 * b])
        pltpu.async_copy(dst2d.at[pl.ds(row0, 3)], dks[b], gi.at[2 * b + 1])

    def wait_block(b):
        pltpu.make_async_copy(src2d.at[pl.ds(0, 3)], sbs[b],
                              gi.at[2 * b]).wait()
        pltpu.make_async_copy(dst2d.at[pl.ds(0, 3)], dks[b],
                              gi.at[2 * b + 1]).wait()

    def start_gathers(t, k, k2, b, r):
        @pl.when(t < ntile)
        def _():
            pltpu.async_copy(ht.at[sbs[b].at[r]], hbs[k], gh.at[k])
            pltpu.async_copy(ed.at[dks[b].at[r]], dbs[k2], ge.at[k2])

    def step(t, k, k2, b, r):
        hb, db = hbs[k], dbs[k2]

        @pl.when(t < ntile)
        def _():
            pltpu.make_async_copy(ht.at[dk0.at[0]], hb, gh.at[k]).wait()
            pltpu.make_async_copy(ed.at[dk0.at[0]], db, ge.at[k2]).wait()

            @plsc.parallel_loop(0, EC, unroll=2)
            def _(e):
                vs = hb[e, pl.ds(F, 16)]
                vd = db[e, :]
                x = vs + vd
                x = jnp.maximum(x, x * 0.2)
                ex = jnp.exp(x)
                hb[e, pl.ds(F, 16)] = ex
                for g in range(H):
                    s = pl.ds(g * 16, 16)
                    hb[e, s] = hb[e, s] * ex

            pltpu.async_copy(hb, acc.at[dks[b].at[r]], ss.at[k], add=True)

        kp2 = (k + 2) % 3

        @pl.when((t >= 1) & (t - 1 < ntile))
        def _():
            pltpu.make_async_copy(hbs[kp2], acc.at[dk0.at[0]],
                                  ss.at[kp2]).wait()

    load_block(0, 0)
    wait_block(0)
    start_gathers(0, 0, 0, 0, 0)
    start_gathers(1, 1, 1, 0, 1)

    def pairbody(u, _):
        t0 = u * 6
        step(t0 + 0, 0, 0, 0, 0)
        load_block(2 * u + 1, 1)
        start_gathers(t0 + 2, 2, 0, 0, 2)
        step(t0 + 1, 1, 1, 0, 1)
        wait_block(1)
        start_gathers(t0 + 3, 0, 1, 1, 0)
        step(t0 + 2, 2, 0, 0, 2)
        start_gathers(t0 + 4, 1, 0, 1, 1)
        step(t0 + 3, 0, 1, 1, 0)
        load_block(2 * u + 2, 0)
        start_gathers(t0 + 5, 2, 1, 1, 2)
        step(t0 + 4, 1, 0, 1, 1)
        wait_block(0)
        start_gathers(t0 + 6, 0, 0, 0, 0)
        step(t0 + 5, 2, 1, 1, 2)
        start_gathers(t0 + 7, 1, 1, 0, 1)
        return 0

    lax.fori_loop(0, NSUP // 2, pairbody, 0)

    plsc.subcore_barrier()
    pltpu.sync_copy(acc.at[pl.ds(sid * 640, 640)],
                    part_out.at[cid, pl.ds(sid * 640, 640)])


def _finish_body(part, out):
    p = part[0] + part[1]
    num = p[:, :F]
    d8 = p[:, F:DA]
    i8 = lax.broadcasted_iota(jnp.int32, (H, F), 0)
    j = lax.broadcasted_iota(jnp.int32, (H, F), 1)
    r8 = (i8 == (j % H)).astype(jnp.float32)
    dd = jnp.dot(d8, r8, preferred_element_type=jnp.float32)
    a = num / (dd + 1e-16)
    jj = lax.broadcasted_iota(jnp.int32, (F, F), 0)
    mm = lax.broadcasted_iota(jnp.int32, (F, F), 1)
    perm = (jj == ((mm % FO) * H + mm // FO)).astype(jnp.float32)
    out[...] = jnp.maximum(jnp.dot(a, perm, preferred_element_type=jnp.float32),
                           0.0)


def kernel(lanes_feat, lane_ids, edge_index, road_emb, lane_emb, W, a_src,
           a_dst):
    f32 = jnp.float32
    lf = lanes_feat.astype(f32)
    wp = W.astype(f32).reshape(F, H, FO).transpose(0, 2, 1).reshape(F, F)
    asf = a_src.astype(f32).transpose(1, 0).reshape(1, F)
    adf = a_dst.astype(f32).transpose(1, 0).reshape(1, F)

    ids = lane_ids.astype(jnp.int32)
    i0 = jnp.zeros((NPAD,), jnp.int32).at[:N].set(ids[:, 0]).reshape(NCH, 64)
    i1 = jnp.zeros((NPAD,), jnp.int32).at[:N].set(ids[:, 1]).reshape(NCH, 64)
    eix = edge_index.astype(jnp.int32).reshape(2 * E // EC, EC)

    fA, rA, lA = pl.pallas_call(
        _dense_body,
        out_shape=[
            jax.ShapeDtypeStruct((NPAD, DT), f32),
            jax.ShapeDtypeStruct((V, DT), f32),
            jax.ShapeDtypeStruct((V, DT), f32),
        ],
    )(lf, road_emb.astype(f32), lane_emb.astype(f32), wp, asf, adf)

    mesh = plsc.VectorSubcoreMesh(core_axis_name="c", subcore_axis_name="s")
    sc_params = pltpu.CompilerParams(use_tc_tiling_on_sc=False)

    node_k = pl.kernel(
        _node_body,
        out_type=[
            jax.ShapeDtypeStruct((NPAD, DH), f32),
            jax.ShapeDtypeStruct((NPAD, L), f32),
            jax.ShapeDtypeStruct((ERPAD, EC), jnp.int32),
            jax.ShapeDtypeStruct((ERPAD, EC), jnp.int32),
        ],
        mesh=mesh,
        compiler_params=sc_params,
        scratch_types=(
            [pltpu.VMEM((64, DT), f32)] * 6
            + [pltpu.VMEM((64,), jnp.int32)] * 4
            + [pltpu.VMEM((64, DH), f32)] * 2
            + [pltpu.VMEM((64, L), f32)] * 2
            + [pltpu.SemaphoreType.DMA((2,))] * 6
        ),
    )
    ht, edt, src2d, dst2d = node_k(fA, rA, lA, i0, i1, eix)

    edge_k = pl.kernel(
        _edge_body,
        out_type=[jax.ShapeDtypeStruct((NC, NPAD, DH), f32)],
        mesh=mesh,
        compiler_params=sc_params,
        scratch_types=(
            [pltpu.VMEM((EC, DH), f32)] * 3
            + [pltpu.VMEM((EC, L), f32)] * 2
            + [pltpu.VMEM((3, EC), jnp.int32)] * 4
            + [pltpu.VMEM_SHARED((NPAD, DH), f32)]
            + [pltpu.SemaphoreType.DMA((3,))]
            + [pltpu.SemaphoreType.DMA((2,))]
            + [pltpu.SemaphoreType.DMA((3,))]
            + [pltpu.SemaphoreType.DMA((4,))]
        ),
    )
    (part,) = edge_k(ht, edt, src2d, dst2d)

    out = pl.pallas_call(
        _finish_body,
        grid=(N // 400,),
        in_specs=[pl.BlockSpec((NC, 400, DH), lambda i: (0, i, 0))],
        out_specs=pl.BlockSpec((400, F), lambda i: (i, 0)),
        out_shape=jax.ShapeDtypeStruct((N, F), f32),
    )(part)
    return out


# speedup vs baseline: 1.2237x; 1.2237x over previous
import jax
import jax.numpy as jnp
from jax import lax
from jax.experimental import pallas as pl
from jax.experimental.pallas import tpu as pltpu
from jax.experimental.pallas import tpu_sc as plsc

N = 10000
E = 320000
L = 16
F = 128
H = 8
FO = 16
V = 1000
NLANES = 16
NC, NS = 2, 16
NPAD = 10240
NCH = NPAD // 64
EC = 80
CPT = 125
ERPAD = 4032
NSUP = 42
DH = F + 16
DA = F + 8
DT = 2 * F


def _dense_body(lf, re_, le, wp, asf, adf, fA, rA, lA):
    w = wp[...]
    a_s = asf[...]
    a_d = adf[...]
    j = lax.broadcasted_iota(jnp.int32, (F, L), 0)
    l = lax.broadcasted_iota(jnp.int32, (F, L), 1)
    b16 = ((j % H) == (l % H)).astype(jnp.float32)
    we_s = jnp.dot((w * a_s), b16, preferred_element_type=jnp.float32)
    we_d = jnp.dot((w * a_d), b16, preferred_element_type=jnp.float32)
    wext = jnp.concatenate(
        [w, we_s, we_d, jnp.zeros((F, DT - F - 2 * L), jnp.float32)], axis=1)
    f = jnp.dot(lf[...], wext[:L - 2], preferred_element_type=jnp.float32)
    fA[...] = jnp.concatenate(
        [f, jnp.zeros((NPAD - N, DT), jnp.float32)], axis=0)
    rA[...] = jnp.dot(re_[...], wext[L - 2:], preferred_element_type=jnp.float32)
    lA[...] = jnp.dot(le[...], wext[L - 2:], preferred_element_type=jnp.float32)


def _node_body(fA, rA, lA, i0_2d, i1_2d, ht_out, ed_out,
               fb0, fb1, rb0, rb1, lb0, lb1, i0b0, i0b1, i1b0, i1b1,
               htb0, htb1, edb0, edb1, gf, gr, gl, wh, we):
    cid = lax.axis_index("c")
    sid = lax.axis_index("s")
    wid = sid * NC + cid
    fbs, rbs, lbs = (fb0, fb1), (rb0, rb1), (lb0, lb1)
    i0s, i1s = (i0b0, i0b1), (i1b0, i1b1)
    hts, eds = (htb0, htb1), (edb0, edb1)

    def issue(t, b):
        c = wid + t * 32
        pltpu.sync_copy(i0_2d.at[c], i0s[b])
        pltpu.sync_copy(i1_2d.at[c], i1s[b])
        pltpu.async_copy(fA.at[pl.ds(c * 64, 64)], fbs[b], gf.at[b])
        pltpu.async_copy(rA.at[i0s[b]], rbs[b], gr.at[b])
        pltpu.async_copy(lA.at[i1s[b]], lbs[b], gl.at[b])

    issue(0, 0)
    for t in range(5):
        b = t % 2
        if t < 4:
            issue(t + 1, 1 - b)
        pltpu.make_async_copy(fA.at[pl.ds(0, 64)], fbs[b], gf.at[b]).wait()
        pltpu.make_async_copy(rA.at[i0s[b]], rbs[b], gr.at[b]).wait()
        pltpu.make_async_copy(lA.at[i1s[b]], lbs[b], gl.at[b]).wait()
        if t >= 2:
            pltpu.make_async_copy(hts[b], ht_out.at[pl.ds(0, 64)],
                                  wh.at[b]).wait()
            pltpu.make_async_copy(eds[b], ed_out.at[pl.ds(0, 64)],
                                  we.at[b]).wait()
        fb, rb, lb, htb, edb = fbs[b], rbs[b], lbs[b], hts[b], eds[b]

        @plsc.parallel_loop(0, 64, unroll=2)
        def _(n):
            for g in range(H):
                s = pl.ds(g * 16, 16)
                htb[n, s] = fb[n, s] + rb[n, s] + lb[n, s]
            se = pl.ds(F, 16)
            htb[n, pl.ds(F, 16)] = fb[n, se] + rb[n, se] + lb[n, se]
            sd = pl.ds(F + 16, 16)
            edb[n, :] = fb[n, sd] + rb[n, sd] + lb[n, sd]

        base = (wid + t * 32) * 64
        pltpu.async_copy(htb, ht_out.at[pl.ds(base, 64)], wh.at[b])
        pltpu.async_copy(edb, ed_out.at[pl.ds(base, 64)], we.at[b])
    for b in range(2):
        pltpu.make_async_copy(hts[b], ht_out.at[pl.ds(0, 64)],
                              wh.at[b]).wait()
        pltpu.make_async_copy(eds[b], ed_out.at[pl.ds(0, 64)],
                              we.at[b]).wait()


def _edge_body(ht, ed, src2d, dst2d, part_out, hb0, hb1, hb2, db0, db1,
               sb0, sb1, dk0, dk1, acc, gh, ge, ss, gi):
    cid = lax.axis_index("c")
    sid = lax.axis_index("s")
    wid = sid * NC + cid
    ntile = jnp.minimum(E // EC - wid * CPT, CPT)
    rowbase = wid * CPT
    hbs = (hb0, hb1, hb2)
    dbs = (db0, db1)
    sbs = (sb0, sb1)
    dks = (dk0, dk1)

    zrow = jnp.zeros((NLANES,), jnp.float32)

    def zero_row(n, _):
        for g in range(DH // 16):
            hb0[n, pl.ds(g * 16, 16)] = zrow
        return 0

    lax.fori_loop(0, EC, zero_row, 0)
    for k in range(NPAD // NS // EC):
        pltpu.sync_copy(hb0, acc.at[pl.ds(sid * 640 + k * EC, EC)])
    plsc.subcore_barrier()

    def load_block(s, b):
        row0 = rowbase + 3 * s
        pltpu.async_copy(src2d.at[pl.ds(row0, 3)], sbs[b], gi.at[2 * b])
        pltpu.async_copy(dst2d.at[pl.ds(row0, 3)], dks[b], gi.at[2 * b + 1])

    def wait_block(b):
        pltpu.make_async_copy(src2d.at[pl.ds(0, 3)], sbs[b],
                              gi.at[2 * b]).wait()
        pltpu.make_async_copy(dst2d.at[pl.ds(0, 3)], dks[b],
                              gi.at[2 * b + 1]).wait()

    def start_gathers(t, k, k2, b, r):
        @pl.when(t < ntile)
        def _():
            pltpu.async_copy(ht.at[sbs[b].at[r]], hbs[k], gh.at[k])
            pltpu.async_copy(ed.at[dks[b].at[r]], dbs[k2], ge.at[k2])

    def step(t, k, k2, b, r):
        hb, db = hbs[k], dbs[k2]

        @pl.when(t < ntile)
        def _():
            pltpu.make_async_copy(ht.at[dk0.at[0]], hb, gh.at[k]).wait()
            pltpu.make_async_copy(ed.at[dk0.at[0]], db, ge.at[k2]).wait()

            @plsc.parallel_loop(0, EC, unroll=2)
            def _(e):
                vs = hb[e, pl.ds(F, 16)]
                vd = db[e, :]
                x = vs + vd
                x = jnp.maximum(x, x * 0.2)
                ex = jnp.exp(x)
                hb[e, pl.ds(F, 16)] = ex
                for g in range(H):
                    s = pl.ds(g * 16, 16)
                    hb[e, s] = hb[e, s] * ex

            pltpu.async_copy(hb, acc.at[dks[b].at[r]], ss.at[k], add=True)

        kp2 = (k + 2) % 3

        @pl.when((t >= 1) & (t - 1 < ntile))
        def _():
            pltpu.make_async_copy(hbs[kp2], acc.at[dk0.at[0]],
                                  ss.at[kp2]).wait()

    load_block(0, 0)
    wait_block(0)
    start_gathers(0, 0, 0, 0, 0)
    start_gathers(1, 1, 1, 0, 1)

    def pairbody(u, _):
        t0 = u * 6
        step(t0 + 0, 0, 0, 0, 0)
        load_block(2 * u + 1, 1)
        start_gathers(t0 + 2, 2, 0, 0, 2)
        step(t0 + 1, 1, 1, 0, 1)
        wait_block(1)
        start_gathers(t0 + 3, 0, 1, 1, 0)
        step(t0 + 2, 2, 0, 0, 2)
        start_gathers(t0 + 4, 1, 0, 1, 1)
        step(t0 + 3, 0, 1, 1, 0)
        load_block(2 * u + 2, 0)
        start_gathers(t0 + 5, 2, 1, 1, 2)
        step(t0 + 4, 1, 0, 1, 1)
        wait_block(0)
        start_gathers(t0 + 6, 0, 0, 0, 0)
        step(t0 + 5, 2, 1, 1, 2)
        start_gathers(t0 + 7, 1, 1, 0, 1)
        return 0

    lax.fori_loop(0, NSUP // 2, pairbody, 0)

    plsc.subcore_barrier()
    pltpu.sync_copy(acc.at[pl.ds(sid * 640, 640)],
                    part_out.at[cid, pl.ds(sid * 640, 640)])


def _finish_body(part, out):
    p = part[0] + part[1]
    num = p[:, :F]
    d8 = p[:, F:DA]
    i8 = lax.broadcasted_iota(jnp.int32, (H, F), 0)
    j = lax.broadcasted_iota(jnp.int32, (H, F), 1)
    r8 = (i8 == (j % H)).astype(jnp.float32)
    dd = jnp.dot(d8, r8, preferred_element_type=jnp.float32)
    a = num / (dd + 1e-16)
    jj = lax.broadcasted_iota(jnp.int32, (F, F), 0)
    mm = lax.broadcasted_iota(jnp.int32, (F, F), 1)
    perm = (jj == ((mm % FO) * H + mm // FO)).astype(jnp.float32)
    out[...] = jnp.maximum(jnp.dot(a, perm, preferred_element_type=jnp.float32),
                           0.0)


def kernel(lanes_feat, lane_ids, edge_index, road_emb, lane_emb, W, a_src,
           a_dst):
    f32 = jnp.float32
    lf = lanes_feat.astype(f32)
    wp = W.astype(f32).reshape(F, H, FO).transpose(0, 2, 1).reshape(F, F)
    asf = a_src.astype(f32).transpose(1, 0).reshape(1, F)
    adf = a_dst.astype(f32).transpose(1, 0).reshape(1, F)

    ids = lane_ids.astype(jnp.int32)
    i0 = jnp.zeros((NPAD,), jnp.int32).at[:N].set(ids[:, 0]).reshape(NCH, 64)
    i1 = jnp.zeros((NPAD,), jnp.int32).at[:N].set(ids[:, 1]).reshape(NCH, 64)
    epad = ERPAD * EC - E
    src2d = jnp.pad(edge_index[0].astype(jnp.int32), (0, epad)).reshape(
        ERPAD, EC)
    dst2d = jnp.pad(edge_index[1].astype(jnp.int32), (0, epad)).reshape(
        ERPAD, EC)

    fA, rA, lA = pl.pallas_call(
        _dense_body,
        out_shape=[
            jax.ShapeDtypeStruct((NPAD, DT), f32),
            jax.ShapeDtypeStruct((V, DT), f32),
            jax.ShapeDtypeStruct((V, DT), f32),
        ],
    )(lf, road_emb.astype(f32), lane_emb.astype(f32), wp, asf, adf)

    mesh = plsc.VectorSubcoreMesh(core_axis_name="c", subcore_axis_name="s")
    sc_params = pltpu.CompilerParams(use_tc_tiling_on_sc=False)

    node_k = pl.kernel(
        _node_body,
        out_type=[
            jax.ShapeDtypeStruct((NPAD, DH), f32),
            jax.ShapeDtypeStruct((NPAD, L), f32),
        ],
        mesh=mesh,
        compiler_params=sc_params,
        scratch_types=(
            [pltpu.VMEM((64, DT), f32)] * 6
            + [pltpu.VMEM((64,), jnp.int32)] * 4
            + [pltpu.VMEM((64, DH), f32)] * 2
            + [pltpu.VMEM((64, L), f32)] * 2
            + [pltpu.SemaphoreType.DMA((2,))] * 5
        ),
    )
    ht, edt = node_k(fA, rA, lA, i0, i1)

    edge_k = pl.kernel(
        _edge_body,
        out_type=[jax.ShapeDtypeStruct((NC, NPAD, DH), f32)],
        mesh=mesh,
        compiler_params=sc_params,
        scratch_types=(
            [pltpu.VMEM((EC, DH), f32)] * 3
            + [pltpu.VMEM((EC, L), f32)] * 2
            + [pltpu.VMEM((3, EC), jnp.int32)] * 4
            + [pltpu.VMEM_SHARED((NPAD, DH), f32)]
            + [pltpu.SemaphoreType.DMA((3,))]
            + [pltpu.SemaphoreType.DMA((2,))]
            + [pltpu.SemaphoreType.DMA((3,))]
            + [pltpu.SemaphoreType.DMA((4,))]
        ),
    )
    (part,) = edge_k(ht, edt, src2d, dst2d)

    out = pl.pallas_call(
        _finish_body,
        grid=(N // 400,),
        in_specs=[pl.BlockSpec((NC, 400, DH), lambda i: (0, i, 0))],
        out_specs=pl.BlockSpec((400, F), lambda i: (i, 0)),
        out_shape=jax.ShapeDtypeStruct((N, F), f32),
    )(part)
    return out
